# Initial kernel scaffold; baseline (speedup 1.0000x reference)
import jax
import jax.numpy as jnp
from jax import lax
from jax.experimental import pallas as pl
from jax.experimental.pallas import tpu as pltpu
from jax.experimental.pallas import tpu_sc as plsc

N_NODES = 10000
N_EDGES = 320000

NC = 2
NS = 16
NW = NC * NS
GRP = 128
GPC = 8
CHUNK = GRP * GPC
E_PER_W = 10240
CHUNKS_PER_W = E_PER_W // CHUNK
E_PAD = NW * E_PER_W
GROWS_PER_W = E_PER_W // GRP
N_ACC = 10240
ROWS_PER_SUB = N_ACC // NS
PAD_SPREAD = 64
DW = 8

_mesh = plsc.VectorSubcoreMesh(core_axis_name="c", subcore_axis_name="s")


def _deg_body(dst_hbm, ones_hbm, zeros_hbm, out_hbm, idx_v, ones_v, acc):
    c = lax.axis_index("c")
    s = lax.axis_index("s")
    wid = s * NC + c
    r0 = s * ROWS_PER_SUB
    pltpu.sync_copy(zeros_hbm.at[pl.ds(r0, ROWS_PER_SUB)],
                    acc.at[pl.ds(r0, ROWS_PER_SUB)])
    pltpu.sync_copy(ones_hbm, ones_v)
    plsc.subcore_barrier()

    @pl.loop(0, CHUNKS_PER_W)
    def _chunk(k):
        row0 = wid * GROWS_PER_W + k * GPC
        pltpu.sync_copy(dst_hbm.at[pl.ds(row0, GPC)], idx_v)
        for j in range(GPC):
            pltpu.sync_copy(ones_v, acc.at[idx_v.at[j]], add=True)

    plsc.subcore_barrier()
    pltpu.sync_copy(acc.at[pl.ds(r0, ROWS_PER_SUB)],
                    out_hbm.at[c, pl.ds(r0, ROWS_PER_SUB)])


_deg_kernel = pl.kernel(
    _deg_body,
    out_type=jax.ShapeDtypeStruct((NC, N_ACC, DW), jnp.float32),
    mesh=_mesh,
    scratch_types=[
        pltpu.VMEM((GPC, GRP), jnp.int32),
        pltpu.VMEM((GRP, DW), jnp.float32),
        pltpu.VMEM_SHARED((N_ACC, DW), jnp.float32),
    ],
)


def _make_spmm(width):
    def body(src_hbm, dst_hbm, g_hbm, zeros_hbm, out_hbm,
             sidx_v, didx_v, rows_v, acc):
        c = lax.axis_index("c")
        s = lax.axis_index("s")
        wid = s * NC + c
        r0 = s * ROWS_PER_SUB
        pltpu.sync_copy(zeros_hbm.at[pl.ds(r0, ROWS_PER_SUB)],
                        acc.at[pl.ds(r0, ROWS_PER_SUB)])
        plsc.subcore_barrier()

        @pl.loop(0, CHUNKS_PER_W)
        def _chunk(k):
            row0 = wid * GROWS_PER_W + k * GPC
            pltpu.sync_copy(src_hbm.at[pl.ds(row0, GPC)], sidx_v)
            pltpu.sync_copy(dst_hbm.at[pl.ds(row0, GPC)], didx_v)
            for j in range(GPC):
                pltpu.sync_copy(g_hbm.at[sidx_v.at[j]], rows_v)
                pltpu.sync_copy(rows_v, acc.at[didx_v.at[j]], add=True)

        plsc.subcore_barrier()
        pltpu.sync_copy(acc.at[pl.ds(r0, ROWS_PER_SUB)],
                        out_hbm.at[c, pl.ds(r0, ROWS_PER_SUB)])

    return pl.kernel(
        body,
        out_type=jax.ShapeDtypeStruct((NC, N_ACC, width), jnp.float32),
        mesh=_mesh,
        scratch_types=[
            pltpu.VMEM((GPC, GRP), jnp.int32),
            pltpu.VMEM((GPC, GRP), jnp.int32),
            pltpu.VMEM((GRP, width), jnp.float32),
            pltpu.VMEM_SHARED((N_ACC, width), jnp.float32),
        ],
    )


_spmm64 = _make_spmm(64)
_spmm32 = _make_spmm(32)


def _tc_matmul1(x_pad, W1):
    def body(x_ref, w_ref, o_ref):
        o_ref[...] = jnp.dot(x_ref[...], w_ref[...],
                             preferred_element_type=jnp.float32)
    return pl.pallas_call(
        body,
        out_shape=jax.ShapeDtypeStruct((N_ACC, 64), jnp.float32),
    )(x_pad, W1)


def _tc_scale(h1, dp):
    def body(h_ref, dp_ref, g_ref, dinv_ref):
        dpv = dp_ref[...]
        deg = dpv[0, :, 0:1] + dpv[1, :, 0:1] + 1.0
        row = lax.broadcasted_iota(jnp.int32, (N_ACC, 1), 0)
        dinv = jnp.where(row < N_NODES, lax.rsqrt(deg), 0.0)
        dinv_ref[...] = dinv
        g_ref[...] = dinv * h_ref[...]
    return pl.pallas_call(
        body,
        out_shape=[
            jax.ShapeDtypeStruct((N_ACC, 64), jnp.float32),
            jax.ShapeDtypeStruct((N_ACC, 1), jnp.float32),
        ],
    )(h1, dp)


def _tc_layer2(p1, g1, dinv, b1, W2):
    def body(p_ref, g_ref, dinv_ref, b_ref, w_ref, o_ref):
        pv = p_ref[...]
        agg = pv[0] + pv[1] + g_ref[...]
        dinv = dinv_ref[...]
        h = jnp.maximum(dinv * agg + b_ref[...], 0.0)
        o_ref[...] = dinv * jnp.dot(h, w_ref[...],
                                    preferred_element_type=jnp.float32)
    return pl.pallas_call(
        body,
        out_shape=jax.ShapeDtypeStruct((N_ACC, 32), jnp.float32),
    )(p1, g1, dinv, b1, W2)


def _tc_head(p2, g2, dinv, b2, Wfc, bfc):
    def body(p_ref, g_ref, dinv_ref, b_ref, w_ref, bf_ref, o_ref):
        pv = p_ref[...]
        agg = pv[0] + pv[1] + g_ref[...]
        h = jnp.maximum(dinv_ref[...] * agg + b_ref[...], 0.0)
        o_ref[...] = jnp.dot(h, w_ref[...],
                             preferred_element_type=jnp.float32) + bf_ref[...]
    return pl.pallas_call(
        body,
        out_shape=jax.ShapeDtypeStruct((N_ACC, 1), jnp.float32),
    )(p2, g2, dinv, b2, Wfc, bfc)


def kernel(x, edge_index, W1, b1, W2, b2, Wfc, bfc):
    src = edge_index[0].astype(jnp.int32)
    dst = edge_index[1].astype(jnp.int32)
    pad = N_NODES + (jnp.arange(E_PAD - N_EDGES, dtype=jnp.int32) % PAD_SPREAD)
    src2 = jnp.concatenate([src, pad]).reshape(E_PAD // GRP, GRP)
    dst2 = jnp.concatenate([dst, pad]).reshape(E_PAD // GRP, GRP)
    x_pad = jnp.pad(x, ((0, N_ACC - N_NODES), (0, 0)))

    ones_dw = jnp.ones((GRP, DW), jnp.float32)
    zeros_dw = jnp.zeros((N_ACC, DW), jnp.float32)
    zeros64 = jnp.zeros((N_ACC, 64), jnp.float32)
    zeros32 = jnp.zeros((N_ACC, 32), jnp.float32)

    dp = _deg_kernel(dst2, ones_dw, zeros_dw)
    h1 = _tc_matmul1(x_pad, W1)
    g1, dinv = _tc_scale(h1, dp)
    p1 = _spmm64(src2, dst2, g1, zeros64)
    g2 = _tc_layer2(p1, g1, dinv, b1.reshape(1, 64), W2)
    p2 = _spmm32(src2, dst2, g2, zeros32)
    out = _tc_head(p2, g2, dinv, b2.reshape(1, 32), Wfc, bfc.reshape(1, 1))
    return out[:N_NODES]


# speedup vs baseline: 27.7581x; 27.7581x over previous
import jax
import jax.numpy as jnp
from jax import lax
from jax.experimental import pallas as pl
from jax.experimental.pallas import tpu as pltpu
from jax.experimental.pallas import tpu_sc as plsc

N_NODES = 10000
N_EDGES = 320000

NC = 2
NS = 16
NW = NC * NS
GRP = 128
GPC = 8
CHUNK = GRP * GPC
E_PER_W = 10240
CHUNKS_PER_W = E_PER_W // CHUNK
E_PAD = NW * E_PER_W
GROWS_PER_W = E_PER_W // GRP
N_ACC = 10240
ROWS_PER_SUB = N_ACC // NS
PAD_SPREAD = 64
DW = 8

_mesh = plsc.VectorSubcoreMesh(core_axis_name="c", subcore_axis_name="s")
_sc_params = pltpu.CompilerParams(use_tc_tiling_on_sc=False)


def _deg_body(dst_hbm, ones_hbm, zeros_hbm, out_hbm, idx_v, ones_v, acc):
    c = lax.axis_index("c")
    s = lax.axis_index("s")
    wid = s * NC + c
    r0 = s * ROWS_PER_SUB
    pltpu.sync_copy(zeros_hbm.at[pl.ds(r0, ROWS_PER_SUB)],
                    acc.at[pl.ds(r0, ROWS_PER_SUB)])
    pltpu.sync_copy(ones_hbm, ones_v)
    plsc.subcore_barrier()

    @pl.loop(0, CHUNKS_PER_W)
    def _chunk(k):
        row0 = wid * GROWS_PER_W + k * GPC
        pltpu.sync_copy(dst_hbm.at[pl.ds(row0, GPC)], idx_v)
        for j in range(GPC):
            pltpu.sync_copy(ones_v, acc.at[idx_v.at[j]], add=True)

    plsc.subcore_barrier()
    pltpu.sync_copy(acc.at[pl.ds(r0, ROWS_PER_SUB)],
                    out_hbm.at[c, pl.ds(r0, ROWS_PER_SUB)])


_deg_kernel = pl.kernel(
    _deg_body,
    out_type=jax.ShapeDtypeStruct((NC, N_ACC, DW), jnp.float32),
    mesh=_mesh,
    compiler_params=_sc_params,
    scratch_types=[
        pltpu.VMEM((GPC, GRP), jnp.int32),
        pltpu.VMEM((GRP, DW), jnp.float32),
        pltpu.VMEM_SHARED((N_ACC, DW), jnp.float32),
    ],
)


def _make_spmm(width):
    def body(src_hbm, dst_hbm, g_hbm, zeros_hbm, out_hbm,
             sidx_v, didx_v, rows_v, acc):
        c = lax.axis_index("c")
        s = lax.axis_index("s")
        wid = s * NC + c
        r0 = s * ROWS_PER_SUB
        pltpu.sync_copy(zeros_hbm.at[pl.ds(r0, ROWS_PER_SUB)],
                        acc.at[pl.ds(r0, ROWS_PER_SUB)])
        plsc.subcore_barrier()

        @pl.loop(0, CHUNKS_PER_W)
        def _chunk(k):
            row0 = wid * GROWS_PER_W + k * GPC
            pltpu.sync_copy(src_hbm.at[pl.ds(row0, GPC)], sidx_v)
            pltpu.sync_copy(dst_hbm.at[pl.ds(row0, GPC)], didx_v)
            for j in range(GPC):
                pltpu.sync_copy(g_hbm.at[sidx_v.at[j]], rows_v)
                pltpu.sync_copy(rows_v, acc.at[didx_v.at[j]], add=True)

        plsc.subcore_barrier()
        pltpu.sync_copy(acc.at[pl.ds(r0, ROWS_PER_SUB)],
                        out_hbm.at[c, pl.ds(r0, ROWS_PER_SUB)])

    return pl.kernel(
        body,
        out_type=jax.ShapeDtypeStruct((NC, N_ACC, width), jnp.float32),
        mesh=_mesh,
        compiler_params=_sc_params,
        scratch_types=[
            pltpu.VMEM((GPC, GRP), jnp.int32),
            pltpu.VMEM((GPC, GRP), jnp.int32),
            pltpu.VMEM((GRP, width), jnp.float32),
            pltpu.VMEM_SHARED((N_ACC, width), jnp.float32),
        ],
    )


_spmm64 = _make_spmm(64)
_spmm32 = _make_spmm(32)


def _tc_matmul1(x_pad, W1):
    def body(x_ref, w_ref, o_ref):
        o_ref[...] = jnp.dot(x_ref[...], w_ref[...],
                             preferred_element_type=jnp.float32)
    return pl.pallas_call(
        body,
        out_shape=jax.ShapeDtypeStruct((N_ACC, 64), jnp.float32),
    )(x_pad, W1)


def _tc_scale(h1, dp):
    def body(h_ref, dp_ref, g_ref, dinv_ref):
        dpv = dp_ref[...]
        deg = dpv[0, :, 0:1] + dpv[1, :, 0:1] + 1.0
        row = lax.broadcasted_iota(jnp.int32, (N_ACC, 1), 0)
        dinv = jnp.where(row < N_NODES, lax.rsqrt(deg), 0.0)
        dinv_ref[...] = dinv
        g_ref[...] = dinv * h_ref[...]
    return pl.pallas_call(
        body,
        out_shape=[
            jax.ShapeDtypeStruct((N_ACC, 64), jnp.float32),
            jax.ShapeDtypeStruct((N_ACC, 1), jnp.float32),
        ],
    )(h1, dp)


def _tc_layer2(p1, g1, dinv, b1, W2):
    def body(p_ref, g_ref, dinv_ref, b_ref, w_ref, o_ref):
        pv = p_ref[...]
        agg = pv[0] + pv[1] + g_ref[...]
        dinv = dinv_ref[...]
        h = jnp.maximum(dinv * agg + b_ref[...], 0.0)
        o_ref[...] = dinv * jnp.dot(h, w_ref[...],
                                    preferred_element_type=jnp.float32)
    return pl.pallas_call(
        body,
        out_shape=jax.ShapeDtypeStruct((N_ACC, 32), jnp.float32),
    )(p1, g1, dinv, b1, W2)


def _tc_head(p2, g2, dinv, b2, Wfc, bfc):
    def body(p_ref, g_ref, dinv_ref, b_ref, w_ref, bf_ref, o_ref):
        pv = p_ref[...]
        agg = pv[0] + pv[1] + g_ref[...]
        h = jnp.maximum(dinv_ref[...] * agg + b_ref[...], 0.0)
        o_ref[...] = jnp.dot(h, w_ref[...],
                             preferred_element_type=jnp.float32) + bf_ref[...]
    return pl.pallas_call(
        body,
        out_shape=jax.ShapeDtypeStruct((N_ACC, 1), jnp.float32),
    )(p2, g2, dinv, b2, Wfc, bfc)


def kernel(x, edge_index, W1, b1, W2, b2, Wfc, bfc):
    src = edge_index[0].astype(jnp.int32)
    dst = edge_index[1].astype(jnp.int32)
    pad = N_NODES + (jnp.arange(E_PAD - N_EDGES, dtype=jnp.int32) % PAD_SPREAD)
    src2 = jnp.concatenate([src, pad]).reshape(E_PAD // GRP, GRP)
    dst2 = jnp.concatenate([dst, pad]).reshape(E_PAD // GRP, GRP)
    x_pad = jnp.pad(x, ((0, N_ACC - N_NODES), (0, 0)))

    ones_dw = jnp.ones((GRP, DW), jnp.float32)
    zeros_dw = jnp.zeros((N_ACC, DW), jnp.float32)
    zeros64 = jnp.zeros((N_ACC, 64), jnp.float32)
    zeros32 = jnp.zeros((N_ACC, 32), jnp.float32)

    dp = _deg_kernel(dst2, ones_dw, zeros_dw)
    h1 = _tc_matmul1(x_pad, W1)
    g1, dinv = _tc_scale(h1, dp)
    p1 = _spmm64(src2, dst2, g1, zeros64)
    g2 = _tc_layer2(p1, g1, dinv, b1.reshape(1, 64), W2)
    p2 = _spmm32(src2, dst2, g2, zeros32)
    out = _tc_head(p2, g2, dinv, b2.reshape(1, 32), Wfc, bfc.reshape(1, 1))
    return out[:N_NODES]


# speedup vs baseline: 40.8869x; 1.4730x over previous
import jax
import jax.numpy as jnp
from jax import lax
from jax.experimental import pallas as pl
from jax.experimental.pallas import tpu as pltpu
from jax.experimental.pallas import tpu_sc as plsc

N_NODES = 10000
N_EDGES = 320000

NC = 2
NS = 16
NW = NC * NS
E_PER_W = 10240
E_PAD = NW * E_PER_W
N_ACC = 10240
ROWS_PER_SUB = N_ACC // NS
PAD_SPREAD = 64
DW = 8

_mesh = plsc.VectorSubcoreMesh(core_axis_name="c", subcore_axis_name="s")
_sc_params = pltpu.CompilerParams(use_tc_tiling_on_sc=False)


def _deg_body(dst_hbm, ones_hbm, zeros_hbm, out_hbm, idx_v, ones_v, acc, sem):
    c = lax.axis_index("c")
    s = lax.axis_index("s")
    wid = s * NC + c
    r0 = s * ROWS_PER_SUB
    pltpu.sync_copy(zeros_hbm.at[pl.ds(r0, ROWS_PER_SUB)],
                    acc.at[pl.ds(r0, ROWS_PER_SUB)])
    pltpu.sync_copy(ones_hbm, ones_v)
    pltpu.sync_copy(dst_hbm.at[wid], idx_v)
    plsc.subcore_barrier()

    @pl.loop(0, 10)
    def _fire(j):
        pltpu.async_copy(ones_v, acc.at[idx_v.at[j]], sem, add=True)

    @pl.loop(0, 10)
    def _drain(j):
        pltpu.make_async_copy(ones_v, acc.at[idx_v.at[0]], sem).wait()

    plsc.subcore_barrier()
    pltpu.sync_copy(acc.at[pl.ds(r0, ROWS_PER_SUB)],
                    out_hbm.at[c, pl.ds(r0, ROWS_PER_SUB)])


_deg_kernel = pl.kernel(
    _deg_body,
    out_type=jax.ShapeDtypeStruct((NC, N_ACC, DW), jnp.float32),
    mesh=_mesh,
    compiler_params=_sc_params,
    scratch_types=[
        pltpu.VMEM((10, 1024), jnp.int32),
        pltpu.VMEM((1024, DW), jnp.float32),
        pltpu.VMEM_SHARED((N_ACC, DW), jnp.float32),
        pltpu.SemaphoreType.DMA,
    ],
)


def _make_spmm(width, rpl):
    nl = E_PER_W // (rpl * 128)
    le = rpl * 128
    assert nl % 2 == 0

    def body(src_hbm, dst_hbm, g_hbm, zeros_hbm, out_hbm,
             sidx, didx, rows0, rows1, acc, gsem0, gsem1, ssem0, ssem1):
        c = lax.axis_index("c")
        s = lax.axis_index("s")
        wid = s * NC + c
        r0 = s * ROWS_PER_SUB
        pltpu.sync_copy(zeros_hbm.at[pl.ds(r0, ROWS_PER_SUB)],
                        acc.at[pl.ds(r0, ROWS_PER_SUB)])
        pltpu.sync_copy(src_hbm.at[wid], sidx)
        pltpu.sync_copy(dst_hbm.at[wid], didx)
        plsc.subcore_barrier()

        def start_g(j, buf, sem):
            pltpu.async_copy(g_hbm.at[sidx.at[j]], buf, sem)

        def wait_g(buf, sem):
            pltpu.make_async_copy(g_hbm.at[sidx.at[0]], buf, sem).wait()

        def start_s(j, buf, sem):
            pltpu.async_copy(buf, acc.at[didx.at[j]], sem, add=True)

        def wait_s(buf, sem):
            pltpu.make_async_copy(buf, acc.at[didx.at[0]], sem).wait()

        start_g(0, rows0, gsem0)
        start_g(1, rows1, gsem1)

        @pl.loop(0, nl // 2)
        def _pipe(p):
            j0 = 2 * p
            wait_g(rows0, gsem0)
            start_s(j0, rows0, ssem0)
            wait_g(rows1, gsem1)
            start_s(j0 + 1, rows1, ssem1)

            @pl.when(p + 1 < nl // 2)
            def _more():
                wait_s(rows0, ssem0)
                start_g(j0 + 2, rows0, gsem0)
                wait_s(rows1, ssem1)
                start_g(j0 + 3, rows1, gsem1)

        wait_s(rows0, ssem0)
        wait_s(rows1, ssem1)
        plsc.subcore_barrier()
        pltpu.sync_copy(acc.at[pl.ds(r0, ROWS_PER_SUB)],
                        out_hbm.at[c, pl.ds(r0, ROWS_PER_SUB)])

    return pl.kernel(
        body,
        out_type=jax.ShapeDtypeStruct((NC, N_ACC, width), jnp.float32),
        mesh=_mesh,
        compiler_params=_sc_params,
        scratch_types=[
            pltpu.VMEM((nl, le), jnp.int32),
            pltpu.VMEM((nl, le), jnp.int32),
            pltpu.VMEM((le, width), jnp.float32),
            pltpu.VMEM((le, width), jnp.float32),
            pltpu.VMEM_SHARED((N_ACC, width), jnp.float32),
            pltpu.SemaphoreType.DMA,
            pltpu.SemaphoreType.DMA,
            pltpu.SemaphoreType.DMA,
            pltpu.SemaphoreType.DMA,
        ],
    )


_spmm64 = _make_spmm(64, 4)
_spmm32 = _make_spmm(32, 8)


def _tc_matmul1(x_pad, W1):
    def body(x_ref, w_ref, o_ref):
        o_ref[...] = jnp.dot(x_ref[...], w_ref[...],
                             preferred_element_type=jnp.float32)
    return pl.pallas_call(
        body,
        out_shape=jax.ShapeDtypeStruct((N_ACC, 64), jnp.float32),
    )(x_pad, W1)


def _tc_scale(h1, dp):
    def body(h_ref, dp_ref, g_ref, dinv_ref):
        dpv = dp_ref[...]
        deg = dpv[0, :, 0:1] + dpv[1, :, 0:1] + 1.0
        row = lax.broadcasted_iota(jnp.int32, (N_ACC, 1), 0)
        dinv = jnp.where(row < N_NODES, lax.rsqrt(deg), 0.0)
        dinv_ref[...] = dinv
        g_ref[...] = dinv * h_ref[...]
    return pl.pallas_call(
        body,
        out_shape=[
            jax.ShapeDtypeStruct((N_ACC, 64), jnp.float32),
            jax.ShapeDtypeStruct((N_ACC, 1), jnp.float32),
        ],
    )(h1, dp)


def _tc_layer2(p1, g1, dinv, b1, W2):
    def body(p_ref, g_ref, dinv_ref, b_ref, w_ref, o_ref):
        pv = p_ref[...]
        agg = pv[0] + pv[1] + g_ref[...]
        dinv = dinv_ref[...]
        h = jnp.maximum(dinv * agg + b_ref[...], 0.0)
        o_ref[...] = dinv * jnp.dot(h, w_ref[...],
                                    preferred_element_type=jnp.float32)
    return pl.pallas_call(
        body,
        out_shape=jax.ShapeDtypeStruct((N_ACC, 32), jnp.float32),
    )(p1, g1, dinv, b1, W2)


def _tc_head(p2, g2, dinv, b2, Wfc, bfc):
    def body(p_ref, g_ref, dinv_ref, b_ref, w_ref, bf_ref, o_ref):
        pv = p_ref[...]
        agg = pv[0] + pv[1] + g_ref[...]
        h = jnp.maximum(dinv_ref[...] * agg + b_ref[...], 0.0)
        o_ref[...] = jnp.dot(h, w_ref[...],
                             preferred_element_type=jnp.float32) + bf_ref[...]
    return pl.pallas_call(
        body,
        out_shape=jax.ShapeDtypeStruct((N_ACC, 1), jnp.float32),
    )(p2, g2, dinv, b2, Wfc, bfc)


def kernel(x, edge_index, W1, b1, W2, b2, Wfc, bfc):
    src = edge_index[0].astype(jnp.int32)
    dst = edge_index[1].astype(jnp.int32)
    pad = N_NODES + (jnp.arange(E_PAD - N_EDGES, dtype=jnp.int32) % PAD_SPREAD)
    src_f = jnp.concatenate([src, pad])
    dst_f = jnp.concatenate([dst, pad])
    src4 = src_f.reshape(NW, 20, 512)
    dst4 = dst_f.reshape(NW, 20, 512)
    src8 = src_f.reshape(NW, 10, 1024)
    dst8 = dst_f.reshape(NW, 10, 1024)
    x_pad = jnp.pad(x, ((0, N_ACC - N_NODES), (0, 0)))

    ones_dw = jnp.ones((1024, DW), jnp.float32)
    zeros_dw = jnp.zeros((N_ACC, DW), jnp.float32)
    zeros64 = jnp.zeros((N_ACC, 64), jnp.float32)
    zeros32 = jnp.zeros((N_ACC, 32), jnp.float32)

    dp = _deg_kernel(dst8, ones_dw, zeros_dw)
    h1 = _tc_matmul1(x_pad, W1)
    g1, dinv = _tc_scale(h1, dp)
    p1 = _spmm64(src4, dst4, g1, zeros64)
    g2 = _tc_layer2(p1, g1, dinv, b1.reshape(1, 64), W2)
    p2 = _spmm32(src8, dst8, g2, zeros32)
    out = _tc_head(p2, g2, dinv, b2.reshape(1, 32), Wfc, bfc.reshape(1, 1))
    return out[:N_NODES]
